# baseline (speedup 1.0000x reference)
import jax
import jax.numpy as jnp
from jax import lax
from jax.experimental import pallas as pl
from jax.experimental.pallas import tpu as pltpu
from jax.experimental.pallas import tpu_sc as plsc

PAD = 1
B, S, H = 4, 4096, 768
L = 16
NC, NS = 2, 16
NW = NC * NS
T = B * S
CHUNK = T // NW
WPR = S // CHUNK
SEG = CHUNK // L
K = 32
KH = K // 2
NSUB = CHUNK // K
NV = H // L
NG = CHUNK // L
G = 8
NB = NV // G


def _mask01(x):
  return jnp.minimum(jnp.abs(x - PAD), 1)


def _body(ids_hbm, tts_hbm, word_hbm, pos_hbm, tok_hbm, lang_hbm, out_hbm,
          pre_v, ids_v, tts_v, pos_v, tmp_v, ftt_v,
          tok0_v, c0_v, d_v,
          w_a, p_a, w_b, p_b, st0, st1,
          gwa, gpa, gwb, gpb, ss0, ss1):
  wid = lax.axis_index("s") * NC + lax.axis_index("c")
  row = wid // WPR
  row_start = row * S
  my_start = wid * CHUNK
  iota = lax.iota(jnp.int32, L)

  def prefix_incl(v):
    acc = v
    for step in (1, 2, 4, 8):
      tmp_v[pl.ds(0, L)] = acc
      sh = plsc.load_gather(tmp_v, [jnp.maximum(iota - step, 0)])
      m = jnp.minimum(jnp.maximum(iota - (step - 1), 0), 1)
      acc = acc + sh * m
    return acc

  def splat_last(v):
    tmp_v[pl.ds(0, L)] = v
    return plsc.load_gather(tmp_v, [jnp.full((L,), L - 1, jnp.int32)])

  pltpu.sync_copy(ids_hbm.at[pl.ds(my_start, CHUNK)], ids_v)
  pltpu.sync_copy(tts_hbm.at[pl.ds(my_start, CHUNK)], tts_v)
  pltpu.async_copy(word_hbm.at[ids_v.at[pl.ds(0, K)]], w_a, gwa)
  pltpu.async_copy(word_hbm.at[ids_v.at[pl.ds(K, K)]], w_b, gwb)

  def pre_block(blk, acc):
    pltpu.sync_copy(ids_hbm.at[pl.ds(row_start + blk * CHUNK, CHUNK)], pre_v)
    def grp(g, a):
      return a + _mask01(pre_v[pl.ds(g * L, L)])
    return lax.fori_loop(0, NG, grp, acc)

  cnt = lax.fori_loop(0, wid % WPR, pre_block, jnp.zeros((L,), jnp.int32))

  def seg_step(t, acc):
    x = plsc.load_gather(ids_v, [iota * SEG + t])
    return acc + _mask01(x)

  seg = lax.fori_loop(0, SEG, seg_step, jnp.zeros((L,), jnp.int32))

  base = splat_last(prefix_incl(cnt))
  lane_base = base + prefix_incl(seg) - seg

  def pos_step(t, acc):
    idx = iota * SEG + t
    mi = _mask01(plsc.load_gather(ids_v, [idx]))
    acc = acc + mi
    plsc.store_scatter(pos_v, [idx], acc * mi + PAD)
    return acc

  lax.fori_loop(0, SEG, pos_step, lane_base)

  pltpu.async_copy(pos_hbm.at[pos_v.at[pl.ds(0, K)]], p_a, gpa)
  pltpu.async_copy(pos_hbm.at[pos_v.at[pl.ds(K, K)]], p_b, gpb)

  pltpu.sync_copy(tok_hbm.at[0], tok0_v)
  pltpu.sync_copy(lang_hbm.at[0], st0.at[0])
  pltpu.sync_copy(lang_hbm.at[1], st0.at[1])

  def mk_const(j, _):
    sl = pl.ds(j * L, L)
    c0_v[sl] = tok0_v[sl] + st0[0, sl]
    d_v[sl] = st0[1, sl] - st0[0, sl]
    return 0

  lax.fori_loop(0, NV, mk_const, 0)

  pairs = ((w_a, p_a, gwa, gpa), (w_b, p_b, gwb, gpb))
  sts = ((st0, ss0), (st1, ss1))

  def pair_iter(pi, _):
    for half, (wb_, pb_, gw, gp) in enumerate(pairs):
      ci = 2 * pi + half
      t0 = ci * K
      pltpu.make_async_copy(word_hbm.at[pl.ds(0, K)], wb_, gw).wait()
      pltpu.make_async_copy(pos_hbm.at[pl.ds(0, K)], pb_, gp).wait()
      for g in range(K // L):
        f16 = tts_v[pl.ds(t0 + g * L, L)].astype(jnp.float32)
        for c in range(L):
          plsc.store_scatter(ftt_v, [iota * L + (g * L * L + c)], f16)
      for sh, (st_, ss) in enumerate(sts):
        for jb in range(NB):
          c0s = [c0_v[pl.ds((jb * G + j) * L, L)] for j in range(G)]
          dds = [d_v[pl.ds((jb * G + j) * L, L)] for j in range(G)]

          def tl(t, _, wb_=wb_, pb_=pb_, st_=st_, sh=sh,
                 c0s=c0s, dds=dds, jb=jb):
            tt_ = t + sh * KH
            f = ftt_v[pl.ds(tt_ * L, L)]
            for j in range(G):
              sl = pl.ds((jb * G + j) * L, L)
              st_[t, sl] = wb_[tt_, sl] + pb_[tt_, sl] + (c0s[j] + f * dds[j])
            return 0

          lax.fori_loop(0, KH, tl, 0)
        def drain(ss=ss, st_=st_):
          pltpu.make_async_copy(st_, out_hbm.at[pl.ds(my_start, KH)],
                                ss).wait()
        pl.when(ci > 0)(drain)
        pltpu.async_copy(st_, out_hbm.at[pl.ds(my_start + t0 + sh * KH, KH)],
                         ss)
      def issue(wb_=wb_, pb_=pb_, gw=gw, gp=gp, t2=t0 + 2 * K):
        pltpu.async_copy(word_hbm.at[ids_v.at[pl.ds(t2, K)]], wb_, gw)
        pltpu.async_copy(pos_hbm.at[pos_v.at[pl.ds(t2, K)]], pb_, gp)
      pl.when(pi < NSUB // 2 - 1)(issue)
    return 0

  lax.fori_loop(0, NSUB // 2, pair_iter, 0)

  pltpu.make_async_copy(st0, out_hbm.at[pl.ds(my_start, KH)], ss0).wait()
  pltpu.make_async_copy(st1, out_hbm.at[pl.ds(my_start, KH)], ss1).wait()


@jax.jit
def _emb(ids, tts, word_emb, pos_emb, tok_type_emb, lang_emb):
  mesh = plsc.VectorSubcoreMesh(core_axis_name="c", subcore_axis_name="s")
  f = pl.kernel(
      _body,
      out_type=jax.ShapeDtypeStruct((T, H), jnp.float32),
      mesh=mesh,
      compiler_params=pltpu.CompilerParams(needs_layout_passes=False),
      scratch_types=[
          pltpu.VMEM((CHUNK,), jnp.int32),
          pltpu.VMEM((CHUNK,), jnp.int32),
          pltpu.VMEM((CHUNK,), jnp.int32),
          pltpu.VMEM((CHUNK,), jnp.int32),
          pltpu.VMEM((L,), jnp.int32),
          pltpu.VMEM((K * L,), jnp.float32),
          pltpu.VMEM((H,), jnp.float32),
          pltpu.VMEM((H,), jnp.float32),
          pltpu.VMEM((H,), jnp.float32),
          pltpu.VMEM((K, H), jnp.float32),
          pltpu.VMEM((K, H), jnp.float32),
          pltpu.VMEM((K, H), jnp.float32),
          pltpu.VMEM((K, H), jnp.float32),
          pltpu.VMEM((KH, H), jnp.float32),
          pltpu.VMEM((KH, H), jnp.float32),
          pltpu.SemaphoreType.DMA,
          pltpu.SemaphoreType.DMA,
          pltpu.SemaphoreType.DMA,
          pltpu.SemaphoreType.DMA,
          pltpu.SemaphoreType.DMA,
          pltpu.SemaphoreType.DMA,
      ],
  )
  return f(ids, tts, word_emb, pos_emb, tok_type_emb, lang_emb)


def kernel(input_ids, token_type_ids, word_emb, pos_emb, tok_type_emb,
           lang_emb):
  ids = input_ids.reshape(-1).astype(jnp.int32)
  tts = token_type_ids.reshape(-1).astype(jnp.int32)
  out = _emb(ids, tts, word_emb, pos_emb, tok_type_emb, lang_emb)
  return out.reshape(B, S, H)


# speedup vs baseline: 1.2815x; 1.2815x over previous
import jax
import jax.numpy as jnp
from jax import lax
from jax.experimental import pallas as pl
from jax.experimental.pallas import tpu as pltpu
from jax.experimental.pallas import tpu_sc as plsc

PAD = 1
B, S, H = 4, 4096, 768
L = 16
NC, NS = 2, 16
NW = NC * NS
T = B * S
CHUNK = T // NW
WPR = S // CHUNK
SEG = CHUNK // L
K = 32
NSUB = CHUNK // K
NV = H // L
NG = CHUNK // L
G = 8
NB = NV // G


def _mask01(x):
  return jnp.minimum(jnp.abs(x - PAD), 1)


def _body(ids_hbm, tts_hbm, word_hbm, pos_hbm, tok_hbm, lang_hbm, out_hbm,
          pre_row, ids_v, tts_v, pos_v, tmp_v, ftt_v,
          tok0_v, lang0_v, lang1_v, c0_v, d_v,
          w_a, p_a, w_b, p_b,
          gwa, gpa, gwb, gpb, ssa, ssb):
  wid = lax.axis_index("s") * NC + lax.axis_index("c")
  row = wid // WPR
  row_start = row * S
  my_start = wid * CHUNK
  iota = lax.iota(jnp.int32, L)

  def prefix_incl(v):
    acc = v
    for step in (1, 2, 4, 8):
      tmp_v[pl.ds(0, L)] = acc
      sh = plsc.load_gather(tmp_v, [jnp.maximum(iota - step, 0)])
      m = jnp.minimum(jnp.maximum(iota - (step - 1), 0), 1)
      acc = acc + sh * m
    return acc

  def splat_last(v):
    tmp_v[pl.ds(0, L)] = v
    return plsc.load_gather(tmp_v, [jnp.full((L,), L - 1, jnp.int32)])

  pltpu.sync_copy(ids_hbm.at[pl.ds(my_start, CHUNK)], ids_v)
  pltpu.sync_copy(tts_hbm.at[pl.ds(my_start, CHUNK)], tts_v)
  pltpu.async_copy(word_hbm.at[ids_v.at[pl.ds(0, K)]], w_a, gwa)
  pltpu.async_copy(word_hbm.at[ids_v.at[pl.ds(K, K)]], w_b, gwb)

  pltpu.sync_copy(ids_hbm.at[pl.ds(row_start, S)], pre_row)

  def pre_grp(g, a):
    return a + _mask01(pre_row[pl.ds(g * L, L)])

  cnt = lax.fori_loop(0, (wid % WPR) * NG, pre_grp,
                      jnp.zeros((L,), jnp.int32))

  def seg_step(t, acc):
    x = plsc.load_gather(ids_v, [iota * SEG + t])
    return acc + _mask01(x)

  seg = lax.fori_loop(0, SEG, seg_step, jnp.zeros((L,), jnp.int32))

  base = splat_last(prefix_incl(cnt))
  lane_base = base + prefix_incl(seg) - seg

  def pos_step(t, acc):
    idx = iota * SEG + t
    mi = _mask01(plsc.load_gather(ids_v, [idx]))
    acc = acc + mi
    plsc.store_scatter(pos_v, [idx], acc * mi + PAD)
    return acc

  lax.fori_loop(0, SEG, pos_step, lane_base)

  pltpu.async_copy(pos_hbm.at[pos_v.at[pl.ds(0, K)]], p_a, gpa)
  pltpu.async_copy(pos_hbm.at[pos_v.at[pl.ds(K, K)]], p_b, gpb)

  pltpu.sync_copy(tok_hbm.at[0], tok0_v)
  pltpu.sync_copy(lang_hbm.at[0], lang0_v)
  pltpu.sync_copy(lang_hbm.at[1], lang1_v)

  def mk_const(j, _):
    sl = pl.ds(j * L, L)
    c0_v[sl] = tok0_v[sl] + lang0_v[sl]
    d_v[sl] = lang1_v[sl] - lang0_v[sl]
    return 0

  lax.fori_loop(0, NV, mk_const, 0)

  pairs = ((w_a, p_a, gwa, gpa, ssa), (w_b, p_b, gwb, gpb, ssb))

  def pair_iter(pi, _):
    for half, (wb_, pb_, gw, gp, ss) in enumerate(pairs):
      t0 = (2 * pi + half) * K
      pltpu.make_async_copy(word_hbm.at[pl.ds(0, K)], wb_, gw).wait()
      pltpu.make_async_copy(pos_hbm.at[pl.ds(0, K)], pb_, gp).wait()
      for g in range(K // L):
        f16 = tts_v[pl.ds(t0 + g * L, L)].astype(jnp.float32)
        for c in range(L):
          plsc.store_scatter(ftt_v, [iota * L + (g * L * L + c)], f16)
      for hh in range(2):
        for jb in range(NB):
          c0s = [c0_v[pl.ds((jb * G + j) * L, L)] for j in range(G)]
          dds = [d_v[pl.ds((jb * G + j) * L, L)] for j in range(G)]

          def tl(t, _, wb_=wb_, pb_=pb_, c0s=c0s, dds=dds, jb=jb, hh=hh):
            tt_ = t + hh * (K // 2)
            f = ftt_v[pl.ds(tt_ * L, L)]
            for j in range(G):
              sl = pl.ds((jb * G + j) * L, L)
              wb_[tt_, sl] = wb_[tt_, sl] + pb_[tt_, sl] + (c0s[j] + f * dds[j])
            return 0

          lax.fori_loop(0, K // 2, tl, 0)
        pltpu.async_copy(
            wb_.at[pl.ds(hh * (K // 2), K // 2)],
            out_hbm.at[pl.ds(my_start + t0 + hh * (K // 2), K // 2)], ss)

    def issue_next():
      for half, (wb_, pb_, gw, gp, ss) in enumerate(pairs):
        t2 = (2 * pi + 2 + half) * K
        pltpu.make_async_copy(wb_, out_hbm.at[pl.ds(my_start, K)], ss).wait()
        pltpu.async_copy(word_hbm.at[ids_v.at[pl.ds(t2, K)]], wb_, gw)
        pltpu.async_copy(pos_hbm.at[pos_v.at[pl.ds(t2, K)]], pb_, gp)

    pl.when(pi < NSUB // 2 - 1)(issue_next)
    return 0

  lax.fori_loop(0, NSUB // 2, pair_iter, 0)

  pltpu.make_async_copy(w_a, out_hbm.at[pl.ds(my_start, K)], ssa).wait()
  pltpu.make_async_copy(w_b, out_hbm.at[pl.ds(my_start, K)], ssb).wait()


@jax.jit
def _emb(ids, tts, word_emb, pos_emb, tok_type_emb, lang_emb):
  mesh = plsc.VectorSubcoreMesh(core_axis_name="c", subcore_axis_name="s")
  f = pl.kernel(
      _body,
      out_type=jax.ShapeDtypeStruct((T, H), jnp.float32),
      mesh=mesh,
      compiler_params=pltpu.CompilerParams(needs_layout_passes=False),
      scratch_types=[
          pltpu.VMEM((S,), jnp.int32),
          pltpu.VMEM((CHUNK,), jnp.int32),
          pltpu.VMEM((CHUNK,), jnp.int32),
          pltpu.VMEM((CHUNK,), jnp.int32),
          pltpu.VMEM((L,), jnp.int32),
          pltpu.VMEM((K * L,), jnp.float32),
          pltpu.VMEM((H,), jnp.float32),
          pltpu.VMEM((H,), jnp.float32),
          pltpu.VMEM((H,), jnp.float32),
          pltpu.VMEM((H,), jnp.float32),
          pltpu.VMEM((H,), jnp.float32),
          pltpu.VMEM((K, H), jnp.float32),
          pltpu.VMEM((K, H), jnp.float32),
          pltpu.VMEM((K, H), jnp.float32),
          pltpu.VMEM((K, H), jnp.float32),
          pltpu.SemaphoreType.DMA,
          pltpu.SemaphoreType.DMA,
          pltpu.SemaphoreType.DMA,
          pltpu.SemaphoreType.DMA,
          pltpu.SemaphoreType.DMA,
          pltpu.SemaphoreType.DMA,
      ],
  )
  return f(ids, tts, word_emb, pos_emb, tok_type_emb, lang_emb)


def kernel(input_ids, token_type_ids, word_emb, pos_emb, tok_type_emb,
           lang_emb):
  ids = input_ids.reshape(-1).astype(jnp.int32)
  tts = token_type_ids.reshape(-1).astype(jnp.int32)
  out = _emb(ids, tts, word_emb, pos_emb, tok_type_emb, lang_emb)
  return out.reshape(B, S, H)


# speedup vs baseline: 1.3905x; 1.0850x over previous
import jax
import jax.numpy as jnp
from jax import lax
from jax.experimental import pallas as pl
from jax.experimental.pallas import tpu as pltpu
from jax.experimental.pallas import tpu_sc as plsc

PAD = 1
B, S, H = 4, 4096, 768
L = 16
NC, NS = 2, 16
NW = NC * NS
T = B * S
CHUNK = T // NW
WPR = S // CHUNK
SEG = CHUNK // L
K = 32
NSUB = CHUNK // K
NV = H // L
NG = CHUNK // L
G = 8
NB = NV // G


def _mask01(x):
  return jnp.minimum(jnp.abs(x - PAD), 1)


def _body(ids_hbm, tts_hbm, word_hbm, pos_hbm, tok_hbm, lang_hbm, out_hbm,
          pre_row, ids_v, tts_v, pos_v, tmp_v, ftt_v,
          tok0_v, lang0_v, lang1_v, c0_v, d_v,
          w_a, p_a, w_b, p_b,
          gwa, gpa, gwb, gpb, ssa, ssb):
  wid = lax.axis_index("s") * NC + lax.axis_index("c")
  row = wid // WPR
  row_start = row * S
  my_start = wid * CHUNK
  iota = lax.iota(jnp.int32, L)

  def prefix_incl(v):
    acc = v
    for step in (1, 2, 4, 8):
      tmp_v[pl.ds(0, L)] = acc
      sh = plsc.load_gather(tmp_v, [jnp.maximum(iota - step, 0)])
      m = jnp.minimum(jnp.maximum(iota - (step - 1), 0), 1)
      acc = acc + sh * m
    return acc

  def splat_last(v):
    tmp_v[pl.ds(0, L)] = v
    return plsc.load_gather(tmp_v, [jnp.full((L,), L - 1, jnp.int32)])

  pltpu.sync_copy(ids_hbm.at[pl.ds(my_start, CHUNK)], ids_v)
  pltpu.sync_copy(tts_hbm.at[pl.ds(my_start, CHUNK)], tts_v)
  pltpu.async_copy(word_hbm.at[ids_v.at[pl.ds(0, K)]], w_a, gwa)
  pltpu.async_copy(word_hbm.at[ids_v.at[pl.ds(K, K)]], w_b, gwb)

  pltpu.sync_copy(ids_hbm.at[pl.ds(row_start, S)], pre_row)

  def pre_grp(g, a):
    return a + _mask01(pre_row[pl.ds(g * L, L)])

  cnt = lax.fori_loop(0, (wid % WPR) * NG, pre_grp,
                      jnp.zeros((L,), jnp.int32))

  def seg_step(t, acc):
    x = plsc.load_gather(ids_v, [iota * SEG + t])
    return acc + _mask01(x)

  seg = lax.fori_loop(0, SEG, seg_step, jnp.zeros((L,), jnp.int32))

  base = splat_last(prefix_incl(cnt))
  lane_base = base + prefix_incl(seg) - seg

  def pos_step(t, acc):
    idx = iota * SEG + t
    mi = _mask01(plsc.load_gather(ids_v, [idx]))
    acc = acc + mi
    plsc.store_scatter(pos_v, [idx], acc * mi + PAD)
    return acc

  lax.fori_loop(0, SEG, pos_step, lane_base)

  pltpu.async_copy(pos_hbm.at[pos_v.at[pl.ds(0, K)]], p_a, gpa)
  pltpu.async_copy(pos_hbm.at[pos_v.at[pl.ds(K, K)]], p_b, gpb)

  pltpu.sync_copy(tok_hbm.at[0], tok0_v)
  pltpu.sync_copy(lang_hbm.at[0], lang0_v)
  pltpu.sync_copy(lang_hbm.at[1], lang1_v)

  def mk_const(j, _):
    sl = pl.ds(j * L, L)
    c0_v[sl] = tok0_v[sl] + lang0_v[sl]
    d_v[sl] = lang1_v[sl] - lang0_v[sl]
    return 0

  lax.fori_loop(0, NV, mk_const, 0)

  pairs = ((w_a, p_a, gwa, gpa, ssa), (w_b, p_b, gwb, gpb, ssb))

  def pair_iter(pi, _):
    for half, (wb_, pb_, gw, gp, ss) in enumerate(pairs):
      t0 = (2 * pi + half) * K
      pltpu.make_async_copy(word_hbm.at[pl.ds(0, K)], wb_, gw).wait()
      pltpu.make_async_copy(pos_hbm.at[pl.ds(0, K)], pb_, gp).wait()
      for g in range(K // L):
        f16 = tts_v[pl.ds(t0 + g * L, L)].astype(jnp.float32)
        for c in range(L):
          plsc.store_scatter(ftt_v, [iota * L + (g * L * L + c)], f16)
      for jb in range(NB):
        c0s = [c0_v[pl.ds((jb * G + j) * L, L)] for j in range(G)]
        dds = [d_v[pl.ds((jb * G + j) * L, L)] for j in range(G)]

        def tl(t, _, wb_=wb_, pb_=pb_, c0s=c0s, dds=dds, jb=jb):
          f = ftt_v[pl.ds(t * L, L)]
          for j in range(G):
            sl = pl.ds((jb * G + j) * L, L)
            wb_[t, sl] = wb_[t, sl] + pb_[t, sl] + (c0s[j] + f * dds[j])
          return 0

        lax.fori_loop(0, K, tl, 0)
      pltpu.async_copy(wb_, out_hbm.at[pl.ds(my_start + t0, K)], ss)

    def issue_next():
      for half, (wb_, pb_, gw, gp, ss) in enumerate(pairs):
        t2 = (2 * pi + 2 + half) * K
        pltpu.make_async_copy(wb_, out_hbm.at[pl.ds(my_start, K)], ss).wait()
        pltpu.async_copy(word_hbm.at[ids_v.at[pl.ds(t2, K)]], wb_, gw)
        pltpu.async_copy(pos_hbm.at[pos_v.at[pl.ds(t2, K)]], pb_, gp)

    pl.when(pi < NSUB // 2 - 1)(issue_next)
    return 0

  lax.fori_loop(0, NSUB // 2, pair_iter, 0)

  pltpu.make_async_copy(w_a, out_hbm.at[pl.ds(my_start, K)], ssa).wait()
  pltpu.make_async_copy(w_b, out_hbm.at[pl.ds(my_start, K)], ssb).wait()


@jax.jit
def _emb(ids, tts, word_emb, pos_emb, tok_type_emb, lang_emb):
  mesh = plsc.VectorSubcoreMesh(core_axis_name="c", subcore_axis_name="s")
  f = pl.kernel(
      _body,
      out_type=jax.ShapeDtypeStruct((T, H), jnp.float32),
      mesh=mesh,
      compiler_params=pltpu.CompilerParams(needs_layout_passes=False),
      scratch_types=[
          pltpu.VMEM((S,), jnp.int32),
          pltpu.VMEM((CHUNK,), jnp.int32),
          pltpu.VMEM((CHUNK,), jnp.int32),
          pltpu.VMEM((CHUNK,), jnp.int32),
          pltpu.VMEM((L,), jnp.int32),
          pltpu.VMEM((K * L,), jnp.float32),
          pltpu.VMEM((H,), jnp.float32),
          pltpu.VMEM((H,), jnp.float32),
          pltpu.VMEM((H,), jnp.float32),
          pltpu.VMEM((H,), jnp.float32),
          pltpu.VMEM((H,), jnp.float32),
          pltpu.VMEM((K, H), jnp.float32),
          pltpu.VMEM((K, H), jnp.float32),
          pltpu.VMEM((K, H), jnp.float32),
          pltpu.VMEM((K, H), jnp.float32),
          pltpu.SemaphoreType.DMA,
          pltpu.SemaphoreType.DMA,
          pltpu.SemaphoreType.DMA,
          pltpu.SemaphoreType.DMA,
          pltpu.SemaphoreType.DMA,
          pltpu.SemaphoreType.DMA,
      ],
  )
  return f(ids, tts, word_emb, pos_emb, tok_type_emb, lang_emb)


def kernel(input_ids, token_type_ids, word_emb, pos_emb, tok_type_emb,
           lang_emb):
  ids = input_ids.reshape(-1).astype(jnp.int32)
  tts = token_type_ids.reshape(-1).astype(jnp.int32)
  out = _emb(ids, tts, word_emb, pos_emb, tok_type_emb, lang_emb)
  return out.reshape(B, S, H)
